# baseline (speedup 1.0000x reference)
import jax
import jax.numpy as jnp
from jax.experimental import pallas as pl

_CHUNK = 512


def _chamfer_body(p_ref, g_ref, p2g_ref, g2p_ref):
    pts = p_ref[0]
    gpts = g_ref[0]
    m = gpts.shape[0]
    px = pts[:, 0:1]
    py = pts[:, 1:2]
    pz = pts[:, 2:3]
    gxr = jnp.transpose(gpts[:, 0:1], (1, 0))
    gyr = jnp.transpose(gpts[:, 1:2], (1, 0))
    gzr = jnp.transpose(gpts[:, 2:3], (1, 0))
    rowmin = None
    g2p_sum = None
    for k in range(0, m, _CHUNK):
        gx = gxr[:, k:k + _CHUNK]
        gy = gyr[:, k:k + _CHUNK]
        gz = gzr[:, k:k + _CHUNK]
        dx = px - gx
        dy = py - gy
        dz = pz - gz
        d = dx * dx + dy * dy + dz * dz
        rm = jnp.min(d, axis=1, keepdims=True)
        rowmin = rm if rowmin is None else jnp.minimum(rowmin, rm)
        cs = jnp.sum(jnp.min(d, axis=0))
        g2p_sum = cs if g2p_sum is None else g2p_sum + cs
    p2g_ref[0] = jnp.sqrt(jnp.mean(rowmin)).reshape(1, 1)
    g2p_ref[0] = jnp.sqrt(g2p_sum / m).reshape(1, 1)


def kernel(points, gts):
    bs, n, _ = points.shape
    m = gts.shape[1]
    p2g_b, g2p_b = pl.pallas_call(
        _chamfer_body,
        grid=(bs,),
        in_specs=[
            pl.BlockSpec((1, n, 3), lambda b: (b, 0, 0)),
            pl.BlockSpec((1, m, 3), lambda b: (b, 0, 0)),
        ],
        out_specs=[
            pl.BlockSpec((1, 1, 1), lambda b: (b, 0, 0)),
            pl.BlockSpec((1, 1, 1), lambda b: (b, 0, 0)),
        ],
        out_shape=[
            jax.ShapeDtypeStruct((bs, 1, 1), jnp.float32),
            jax.ShapeDtypeStruct((bs, 1, 1), jnp.float32),
        ],
    )(points, gts)
    p2g = jnp.mean(p2g_b)
    g2p = jnp.mean(g2p_b)
    loss = (p2g + g2p) / 2.0
    return (loss, p2g, g2p)
